# baseline (speedup 1.0000x reference)
import functools

import jax
import jax.numpy as jnp
from jax import lax
from jax.experimental import pallas as pl
from jax.experimental.pallas import tpu as pltpu
from jax.experimental.pallas import tpu_sc as plsc

SLABS = 4
SLAB_W = 128
CHUNK = 128


def _sc_mesh():
    return plsc.VectorSubcoreMesh(core_axis_name="c", subcore_axis_name="s")


def _sc_layer_pass(init, table, src4, dst3, nacc, nchunk):
    rows_per_tile = nacc // 16

    @functools.partial(
        pl.kernel,
        out_type=jax.ShapeDtypeStruct((SLABS, nacc, SLAB_W), jnp.float32),
        mesh=_sc_mesh(),
        scratch_types=[
            pltpu.VMEM((CHUNK,), jnp.int32),
            pltpu.VMEM((CHUNK,), jnp.int32),
            pltpu.VMEM((CHUNK, SLAB_W), jnp.float32),
            pltpu.VMEM_SHARED((nacc, SLAB_W), jnp.float32),
            pltpu.SemaphoreType.DMA,
        ],
    )
    def k(init_h, table_h, src_h, dst_h, out_h, sidx, didx, gbuf, acc, sem):
        cid = lax.axis_index("c")
        sid = lax.axis_index("s")
        row0 = sid * rows_per_tile
        for t in range(2):
            slab = cid * 2 + t
            pltpu.sync_copy(init_h.at[slab, pl.ds(row0, rows_per_tile)],
                            acc.at[pl.ds(row0, rows_per_tile)])
            plsc.subcore_barrier()

            def body(j, _):
                pltpu.sync_copy(src_h.at[slab, sid, j], sidx)
                pltpu.sync_copy(dst_h.at[sid, j], didx)
                pltpu.async_copy(table_h.at[sidx], gbuf, sem).wait()
                pltpu.sync_copy(gbuf, acc.at[didx], add=True)
                return 0

            lax.fori_loop(0, nchunk, body, 0)
            plsc.subcore_barrier()
            pltpu.sync_copy(acc.at[pl.ds(row0, rows_per_tile)],
                            out_h.at[slab, pl.ds(row0, rows_per_tile)])
            plsc.subcore_barrier()

    return k(init, table, src4, dst3)


def _sc_rel_pass(init, table, rel4, dsto3, dsts3, nacc, nchunk):
    rows_per_tile = nacc // 16

    @functools.partial(
        pl.kernel,
        out_type=jax.ShapeDtypeStruct((SLABS, nacc, SLAB_W), jnp.float32),
        mesh=_sc_mesh(),
        scratch_types=[
            pltpu.VMEM((CHUNK,), jnp.int32),
            pltpu.VMEM((CHUNK,), jnp.int32),
            pltpu.VMEM((CHUNK,), jnp.int32),
            pltpu.VMEM((CHUNK, SLAB_W), jnp.float32),
            pltpu.VMEM_SHARED((nacc, SLAB_W), jnp.float32),
            pltpu.SemaphoreType.DMA,
        ],
    )
    def k(init_h, table_h, rel_h, dsto_h, dsts_h, out_h,
          sidx, didxo, didxs, gbuf, acc, sem):
        cid = lax.axis_index("c")
        sid = lax.axis_index("s")
        row0 = sid * rows_per_tile
        for t in range(2):
            slab = cid * 2 + t
            pltpu.sync_copy(init_h.at[slab, pl.ds(row0, rows_per_tile)],
                            acc.at[pl.ds(row0, rows_per_tile)])
            plsc.subcore_barrier()

            def body(j, _):
                pltpu.sync_copy(rel_h.at[slab, sid, j], sidx)
                pltpu.sync_copy(dsto_h.at[sid, j], didxo)
                pltpu.sync_copy(dsts_h.at[sid, j], didxs)
                pltpu.async_copy(table_h.at[sidx], gbuf, sem).wait()
                pltpu.sync_copy(gbuf, acc.at[didxo], add=True)
                pltpu.sync_copy(gbuf, acc.at[didxs], add=True)
                return 0

            lax.fori_loop(0, nchunk, body, 0)
            plsc.subcore_barrier()
            pltpu.sync_copy(acc.at[pl.ds(row0, rows_per_tile)],
                            out_h.at[slab, pl.ds(row0, rows_per_tile)])
            plsc.subcore_barrier()

    return k(init, table, rel4, dsto3, dsts3)


def _proj(x, w, b, bm):
    m, d = x.shape
    h = w.shape[1]

    def kern(x_ref, w_ref, b_ref, o_ref):
        o_ref[...] = (jnp.dot(x_ref[...], w_ref[...],
                              preferred_element_type=jnp.float32) + b_ref[...])

    return pl.pallas_call(
        kern,
        grid=(m // bm,),
        in_specs=[pl.BlockSpec((bm, d), lambda i: (i, 0)),
                  pl.BlockSpec((d, h), lambda i: (0, 0)),
                  pl.BlockSpec((1, h), lambda i: (0, 0))],
        out_specs=pl.BlockSpec((bm, h), lambda i: (i, 0)),
        out_shape=jax.ShapeDtypeStruct((m, h), jnp.float32),
    )(x, w, b.reshape(1, h))


def _layer_pre(ns, wt, bp, bm):
    m, h = ns.shape

    def kern(ns_ref, wt_ref, bp_ref, o_ref):
        o_ref[...] = jnp.dot(ns_ref[...], wt_ref[...],
                             preferred_element_type=jnp.float32) + bp_ref[...]

    return pl.pallas_call(
        kern,
        grid=(m // bm,),
        in_specs=[pl.BlockSpec((bm, h), lambda i: (i, 0)),
                  pl.BlockSpec((h, h), lambda i: (0, 0)),
                  pl.BlockSpec((1, h), lambda i: (0, 0))],
        out_specs=pl.BlockSpec((bm, h), lambda i: (i, 0)),
        out_shape=jax.ShapeDtypeStruct((m, h), jnp.float32),
    )(ns, wt, bp.reshape(1, h))


def _layer_update(ns, pre, agg, wb, bm):
    m, h = ns.shape

    def kern(ns_ref, pre_ref, agg_ref, wb_ref, o_ref):
        acc = pre_ref[...]
        for s in range(SLABS):
            acc += jnp.dot(agg_ref[s], wb_ref[s],
                           preferred_element_type=jnp.float32)
        o_ref[...] = ns_ref[...] + acc * jax.nn.sigmoid(acc)

    return pl.pallas_call(
        kern,
        grid=(m // bm,),
        in_specs=[pl.BlockSpec((bm, h), lambda i: (i, 0)),
                  pl.BlockSpec((bm, h), lambda i: (i, 0)),
                  pl.BlockSpec((SLABS, bm, SLAB_W), lambda i: (0, i, 0)),
                  pl.BlockSpec((SLABS, SLAB_W, h), lambda i: (0, 0, 0))],
        out_specs=pl.BlockSpec((bm, h), lambda i: (i, 0)),
        out_shape=jax.ShapeDtypeStruct((m, h), jnp.float32),
    )(ns, pre, agg, wb)


def _finalize(ns, img3, g, b, nimg, bm):
    m, h = ns.shape
    grid = m // bm

    def kern(ns_ref, img_ref, g_ref, b_ref, ln_ref, glob_ref, acc_s, acc_c):
        i = pl.program_id(0)
        x = ns_ref[...]
        mu = jnp.mean(x, axis=1, keepdims=True)
        var = jnp.mean((x - mu) ** 2, axis=1, keepdims=True)
        ln = (x - mu) / jnp.sqrt(var + 1e-5) * g_ref[...] + b_ref[...]
        ln_ref[...] = ln
        ids = img_ref[0, 0, :]
        onehot = (ids[None, :] ==
                  lax.broadcasted_iota(jnp.int32, (nimg, bm), 0)
                  ).astype(jnp.float32)

        @pl.when(i == 0)
        def _():
            acc_s[...] = jnp.zeros_like(acc_s)
            acc_c[...] = jnp.zeros_like(acc_c)

        acc_s[...] += jnp.dot(onehot, ln, preferred_element_type=jnp.float32)
        acc_c[...] += jnp.dot(onehot, jnp.ones((bm, h), jnp.float32),
                              preferred_element_type=jnp.float32)
        glob_ref[...] = acc_s[...] / jnp.maximum(acc_c[...], 1.0)

    return pl.pallas_call(
        kern,
        grid=(grid,),
        in_specs=[pl.BlockSpec((bm, h), lambda i: (i, 0)),
                  pl.BlockSpec((1, 1, bm), lambda i: (i, 0, 0)),
                  pl.BlockSpec((1, h), lambda i: (0, 0)),
                  pl.BlockSpec((1, h), lambda i: (0, 0))],
        out_specs=[pl.BlockSpec((bm, h), lambda i: (i, 0)),
                   pl.BlockSpec((nimg, h), lambda i: (0, 0))],
        out_shape=[jax.ShapeDtypeStruct((m, h), jnp.float32),
                   jax.ShapeDtypeStruct((nimg, h), jnp.float32)],
        scratch_shapes=[pltpu.VMEM((nimg, h), jnp.float32),
                        pltpu.VMEM((nimg, h), jnp.float32)],
    )(ns, img3, g.reshape(1, h), b.reshape(1, h))


def _to_slab_major(x):
    n, h = x.shape
    return jnp.reshape(jnp.transpose(jnp.reshape(x, (n, SLABS, SLAB_W)),
                                     (1, 0, 2)), (SLABS * n, SLAB_W))


def _pad_tiles(x, cap, fill):
    return jnp.concatenate(
        [x, jnp.full((cap - x.shape[0],), fill, jnp.int32)]
    ).reshape(16, -1, CHUNK)


def kernel(node_feats, rel_feats, triples, obj_to_img,
           W_node_in, b_node_in, W_rel_in, b_rel_in,
           Wp0, bp0, Wp1, bp1, Wp2, bp2,
           g_node, b_node, g_rel, b_rel):
    n, d = node_feats.shape
    r = rel_feats.shape[0]
    h = W_node_in.shape[1]
    e = triples.shape[0]
    nimg = 64
    bm = 400
    assert h == SLABS * SLAB_W and n % bm == 0

    subj, rel, obj = triples[:, 0], triples[:, 1], triples[:, 2]
    src = jnp.concatenate([subj, obj])
    dst = jnp.concatenate([obj, subj])

    m = 2 * e
    nchunk = -(-(-(-m // 16)) // CHUNK)
    mp = 16 * nchunk * CHUNK
    nchunk_e = -(-(-(-e // 16)) // CHUNK)
    ep = 16 * nchunk_e * CHUNK
    nacc = -(-(n + 1) // 128) * 128
    offs = jnp.arange(SLABS, dtype=jnp.int32)[:, None, None, None]

    dst3 = _pad_tiles(dst, mp, n)
    src4 = _pad_tiles(src, mp, 0)[None] + offs * n
    rel4 = _pad_tiles(rel, ep, 0)[None] + offs * r
    dsto3 = _pad_tiles(obj, ep, n)
    dsts3 = _pad_tiles(subj, ep, n)

    ns = _proj(node_feats, W_node_in, b_node_in, bm)
    rs = _proj(rel_feats, W_rel_in, b_rel_in, bm)

    zeros_init = jnp.zeros((SLABS, nacc, SLAB_W), jnp.float32)
    kagg = _sc_rel_pass(zeros_init, _to_slab_major(rs), rel4, dsto3, dsts3,
                        nacc, nchunk_e)

    for wp, bp in ((Wp0, bp0), (Wp1, bp1), (Wp2, bp2)):
        pre = _layer_pre(ns, wp[:h], bp, bm)
        agg = _sc_layer_pass(kagg, _to_slab_major(ns), src4, dst3,
                             nacc, nchunk)
        wb = wp[h:].reshape(SLABS, SLAB_W, h)
        ns = _layer_update(ns, pre, agg, wb, bm)

    img3 = obj_to_img.astype(jnp.int32).reshape(n // bm, 1, bm)
    ln_ns, glob = _finalize(ns, img3, g_node, b_node, nimg, bm)
    return (ln_ns, glob)


# speedup vs baseline: 1.0051x; 1.0051x over previous
import functools

import jax
import jax.numpy as jnp
from jax import lax
from jax.experimental import pallas as pl
from jax.experimental.pallas import tpu as pltpu
from jax.experimental.pallas import tpu_sc as plsc

SLABS = 4
SLAB_W = 128
CHUNK = 256


def _sc_mesh():
    return plsc.VectorSubcoreMesh(core_axis_name="c", subcore_axis_name="s")


def _sc_layer_pass(init, table, src4, dst3, nacc, nchunk):
    rows_per_tile = nacc // 16

    @functools.partial(
        pl.kernel,
        out_type=jax.ShapeDtypeStruct((SLABS, nacc, SLAB_W), jnp.float32),
        mesh=_sc_mesh(),
        scratch_types=[
            pltpu.VMEM((CHUNK,), jnp.int32),
            pltpu.VMEM((CHUNK,), jnp.int32),
            pltpu.VMEM((CHUNK, SLAB_W), jnp.float32),
            pltpu.VMEM_SHARED((nacc, SLAB_W), jnp.float32),
            pltpu.SemaphoreType.DMA,
        ],
    )
    def k(init_h, table_h, src_h, dst_h, out_h, sidx, didx, gbuf, acc, sem):
        cid = lax.axis_index("c")
        sid = lax.axis_index("s")
        row0 = sid * rows_per_tile
        for t in range(2):
            slab = cid * 2 + t
            pltpu.sync_copy(init_h.at[slab, pl.ds(row0, rows_per_tile)],
                            acc.at[pl.ds(row0, rows_per_tile)])
            plsc.subcore_barrier()

            def body(j, _):
                pltpu.sync_copy(src_h.at[slab, sid, j], sidx)
                pltpu.sync_copy(dst_h.at[sid, j], didx)
                pltpu.async_copy(table_h.at[sidx], gbuf, sem).wait()
                pltpu.sync_copy(gbuf, acc.at[didx], add=True)
                return 0

            lax.fori_loop(0, nchunk, body, 0)
            plsc.subcore_barrier()
            pltpu.sync_copy(acc.at[pl.ds(row0, rows_per_tile)],
                            out_h.at[slab, pl.ds(row0, rows_per_tile)])
            plsc.subcore_barrier()

    return k(init, table, src4, dst3)


def _sc_rel_pass(init, table, rel4, dsto3, dsts3, nacc, nchunk):
    rows_per_tile = nacc // 16

    @functools.partial(
        pl.kernel,
        out_type=jax.ShapeDtypeStruct((SLABS, nacc, SLAB_W), jnp.float32),
        mesh=_sc_mesh(),
        scratch_types=[
            pltpu.VMEM((CHUNK,), jnp.int32),
            pltpu.VMEM((CHUNK,), jnp.int32),
            pltpu.VMEM((CHUNK,), jnp.int32),
            pltpu.VMEM((CHUNK, SLAB_W), jnp.float32),
            pltpu.VMEM_SHARED((nacc, SLAB_W), jnp.float32),
            pltpu.SemaphoreType.DMA,
        ],
    )
    def k(init_h, table_h, rel_h, dsto_h, dsts_h, out_h,
          sidx, didxo, didxs, gbuf, acc, sem):
        cid = lax.axis_index("c")
        sid = lax.axis_index("s")
        row0 = sid * rows_per_tile
        for t in range(2):
            slab = cid * 2 + t
            pltpu.sync_copy(init_h.at[slab, pl.ds(row0, rows_per_tile)],
                            acc.at[pl.ds(row0, rows_per_tile)])
            plsc.subcore_barrier()

            def body(j, _):
                pltpu.sync_copy(rel_h.at[slab, sid, j], sidx)
                pltpu.sync_copy(dsto_h.at[sid, j], didxo)
                pltpu.sync_copy(dsts_h.at[sid, j], didxs)
                pltpu.async_copy(table_h.at[sidx], gbuf, sem).wait()
                pltpu.sync_copy(gbuf, acc.at[didxo], add=True)
                pltpu.sync_copy(gbuf, acc.at[didxs], add=True)
                return 0

            lax.fori_loop(0, nchunk, body, 0)
            plsc.subcore_barrier()
            pltpu.sync_copy(acc.at[pl.ds(row0, rows_per_tile)],
                            out_h.at[slab, pl.ds(row0, rows_per_tile)])
            plsc.subcore_barrier()

    return k(init, table, rel4, dsto3, dsts3)


def _proj(x, w, b, bm):
    m, d = x.shape
    h = w.shape[1]

    def kern(x_ref, w_ref, b_ref, o_ref):
        o_ref[...] = (jnp.dot(x_ref[...], w_ref[...],
                              preferred_element_type=jnp.float32) + b_ref[...])

    return pl.pallas_call(
        kern,
        grid=(m // bm,),
        in_specs=[pl.BlockSpec((bm, d), lambda i: (i, 0)),
                  pl.BlockSpec((d, h), lambda i: (0, 0)),
                  pl.BlockSpec((1, h), lambda i: (0, 0))],
        out_specs=pl.BlockSpec((bm, h), lambda i: (i, 0)),
        out_shape=jax.ShapeDtypeStruct((m, h), jnp.float32),
    )(x, w, b.reshape(1, h))


def _layer_update(ns, agg, wt, wb, bp, bm):
    m, h = ns.shape

    def kern(ns_ref, agg_ref, wt_ref, wb_ref, bp_ref, o_ref):
        acc = jnp.dot(ns_ref[...], wt_ref[...],
                      preferred_element_type=jnp.float32)
        for s in range(SLABS):
            acc += jnp.dot(agg_ref[s], wb_ref[s],
                           preferred_element_type=jnp.float32)
        acc += bp_ref[...]
        o_ref[...] = ns_ref[...] + acc * jax.nn.sigmoid(acc)

    return pl.pallas_call(
        kern,
        grid=(m // bm,),
        in_specs=[pl.BlockSpec((bm, h), lambda i: (i, 0)),
                  pl.BlockSpec((SLABS, bm, SLAB_W), lambda i: (0, i, 0)),
                  pl.BlockSpec((h, h), lambda i: (0, 0)),
                  pl.BlockSpec((SLABS, SLAB_W, h), lambda i: (0, 0, 0)),
                  pl.BlockSpec((1, h), lambda i: (0, 0))],
        out_specs=pl.BlockSpec((bm, h), lambda i: (i, 0)),
        out_shape=jax.ShapeDtypeStruct((m, h), jnp.float32),
    )(ns, agg, wt, wb, bp.reshape(1, h))


def _finalize(ns, img3, g, b, nimg, bm):
    m, h = ns.shape
    grid = m // bm

    def kern(ns_ref, img_ref, g_ref, b_ref, ln_ref, glob_ref, acc_s, acc_c):
        i = pl.program_id(0)
        x = ns_ref[...]
        mu = jnp.mean(x, axis=1, keepdims=True)
        var = jnp.mean((x - mu) ** 2, axis=1, keepdims=True)
        ln = (x - mu) / jnp.sqrt(var + 1e-5) * g_ref[...] + b_ref[...]
        ln_ref[...] = ln
        ids = img_ref[0, 0, :]
        onehot = (ids[None, :] ==
                  lax.broadcasted_iota(jnp.int32, (nimg, bm), 0)
                  ).astype(jnp.float32)

        @pl.when(i == 0)
        def _():
            acc_s[...] = jnp.zeros_like(acc_s)
            acc_c[...] = jnp.zeros_like(acc_c)

        acc_s[...] += jnp.dot(onehot, ln, preferred_element_type=jnp.float32)
        acc_c[...] += jnp.dot(onehot, jnp.ones((bm, h), jnp.float32),
                              preferred_element_type=jnp.float32)
        glob_ref[...] = acc_s[...] / jnp.maximum(acc_c[...], 1.0)

    return pl.pallas_call(
        kern,
        grid=(grid,),
        in_specs=[pl.BlockSpec((bm, h), lambda i: (i, 0)),
                  pl.BlockSpec((1, 1, bm), lambda i: (i, 0, 0)),
                  pl.BlockSpec((1, h), lambda i: (0, 0)),
                  pl.BlockSpec((1, h), lambda i: (0, 0))],
        out_specs=[pl.BlockSpec((bm, h), lambda i: (i, 0)),
                   pl.BlockSpec((nimg, h), lambda i: (0, 0))],
        out_shape=[jax.ShapeDtypeStruct((m, h), jnp.float32),
                   jax.ShapeDtypeStruct((nimg, h), jnp.float32)],
        scratch_shapes=[pltpu.VMEM((nimg, h), jnp.float32),
                        pltpu.VMEM((nimg, h), jnp.float32)],
    )(ns, img3, g.reshape(1, h), b.reshape(1, h))


def _to_slab_major(x):
    n, h = x.shape
    return jnp.reshape(jnp.transpose(jnp.reshape(x, (n, SLABS, SLAB_W)),
                                     (1, 0, 2)), (SLABS * n, SLAB_W))


def _pad_tiles(x, cap, fill):
    return jnp.concatenate(
        [x, jnp.full((cap - x.shape[0],), fill, jnp.int32)]
    ).reshape(16, -1, CHUNK)


def kernel(node_feats, rel_feats, triples, obj_to_img,
           W_node_in, b_node_in, W_rel_in, b_rel_in,
           Wp0, bp0, Wp1, bp1, Wp2, bp2,
           g_node, b_node, g_rel, b_rel):
    n, d = node_feats.shape
    r = rel_feats.shape[0]
    h = W_node_in.shape[1]
    e = triples.shape[0]
    nimg = 64
    bm = 400
    assert h == SLABS * SLAB_W and n % bm == 0

    subj, rel, obj = triples[:, 0], triples[:, 1], triples[:, 2]
    src = jnp.concatenate([subj, obj])
    dst = jnp.concatenate([obj, subj])

    m = 2 * e
    nchunk = -(-(-(-m // 16)) // CHUNK)
    mp = 16 * nchunk * CHUNK
    nchunk_e = -(-(-(-e // 16)) // CHUNK)
    ep = 16 * nchunk_e * CHUNK
    nacc = -(-(n + 1) // 128) * 128
    offs = jnp.arange(SLABS, dtype=jnp.int32)[:, None, None, None]

    dst3 = _pad_tiles(dst, mp, n)
    src4 = _pad_tiles(src, mp, 0)[None] + offs * n
    rel4 = _pad_tiles(rel, ep, 0)[None] + offs * r
    dsto3 = _pad_tiles(obj, ep, n)
    dsts3 = _pad_tiles(subj, ep, n)

    ns = _proj(node_feats, W_node_in, b_node_in, bm)
    rs = _proj(rel_feats, W_rel_in, b_rel_in, bm)

    zeros_init = jnp.zeros((SLABS, nacc, SLAB_W), jnp.float32)
    kagg = _sc_rel_pass(zeros_init, _to_slab_major(rs), rel4, dsto3, dsts3,
                        nacc, nchunk_e)

    for wp, bp in ((Wp0, bp0), (Wp1, bp1), (Wp2, bp2)):
        agg = _sc_layer_pass(kagg, _to_slab_major(ns), src4, dst3,
                             nacc, nchunk)
        wt = wp[:h]
        wb = wp[h:].reshape(SLABS, SLAB_W, h)
        ns = _layer_update(ns, agg, wt, wb, bp, bm)

    img3 = obj_to_img.astype(jnp.int32).reshape(n // bm, 1, bm)
    ln_ns, glob = _finalize(ns, img3, g_node, b_node, nimg, bm)
    return (ln_ns, glob)
